# baseline (speedup 1.0000x reference)
import functools

import jax
import jax.numpy as jnp
from jax import lax
from jax.experimental import pallas as pl
from jax.experimental.pallas import tpu as pltpu
from jax.experimental.pallas import tpu_sc as plsc

N = 10000
E = 320000
SLOPE = 0.2
EPS = 1e-9

NC = 2
NS = 16
NW = NC * NS
EPT = E // NW
C = 80
CHUNKS = EPT // C
NPAD = 10240
NPT = NPAD // NS

NR = 2
NI = 4


def _embed(featp, el, odim):
    col = lax.broadcasted_iota(jnp.int32, featp.shape, 1)
    return jnp.where(col == odim, 1.0, jnp.where(col == odim + 1, el, featp))


def _dense_body(h_ref, wp_ref, alp_ref, arp_ref, featp_ref, er_ref, *, odim):
    featp = jnp.dot(h_ref[...], wp_ref[...], preferred_element_type=jnp.float32)
    el = jnp.sum(featp * alp_ref[...], axis=1, keepdims=True)
    er_ref[...] = jnp.sum(featp * arp_ref[...], axis=1, keepdims=True)
    featp_ref[...] = _embed(featp, el, odim)


def _combine_dense_body(vout_ref, wp_ref, alp_ref, arp_ref,
                        featp_ref, er_ref, *, prev, odim):
    tot = vout_ref[0][:N] + vout_ref[1][:N]
    v = tot[:, :prev]
    s = tot[:, prev:prev + 1]
    h = jnp.maximum(v / (s + EPS), 0.0)
    featp = jnp.dot(h, wp_ref[...], preferred_element_type=jnp.float32)
    el = jnp.sum(featp * alp_ref[...], axis=1, keepdims=True)
    er_ref[...] = jnp.sum(featp * arp_ref[...], axis=1, keepdims=True)
    featp_ref[...] = _embed(featp, el, odim)


def _final_body(vout_ref, out_ref, *, odim):
    tot = vout_ref[0][:N] + vout_ref[1][:N]
    out_ref[...] = tot[:, :odim] / (tot[:, odim:odim + 1] + EPS)


def _dense(h, Wp, alp, arp, odim, dp):
    return pl.pallas_call(
        functools.partial(_dense_body, odim=odim),
        out_shape=[
            jax.ShapeDtypeStruct((N, dp), jnp.float32),
            jax.ShapeDtypeStruct((N, 1), jnp.float32),
        ],
    )(h, Wp, alp, arp)


def _combine_dense(vout, Wp, alp, arp, prev, odim, dp):
    return pl.pallas_call(
        functools.partial(_combine_dense_body, prev=prev, odim=odim),
        out_shape=[
            jax.ShapeDtypeStruct((N, dp), jnp.float32),
            jax.ShapeDtypeStruct((N, 1), jnp.float32),
        ],
    )(vout, Wp, alp, arp)


def _final(vout, odim):
    return pl.pallas_call(
        functools.partial(_final_body, odim=odim),
        out_shape=jax.ShapeDtypeStruct((N, odim), jnp.float32),
    )(vout)


@functools.lru_cache(maxsize=None)
def _make_edge_kernel(dp, odim):
    kv = dp // 16
    mesh = plsc.VectorSubcoreMesh(core_axis_name="c", subcore_axis_name="s")

    @functools.partial(
        pl.kernel,
        out_type=jax.ShapeDtypeStruct((NC, NPAD, dp), jnp.float32),
        mesh=mesh,
        scratch_types=[
            pltpu.VMEM((NI, 2, C), jnp.int32),
            pltpu.VMEM((N,), jnp.float32),
            pltpu.VMEM((C,), jnp.float32),
            pltpu.VMEM((NR, C, dp), jnp.float32),
            pltpu.VMEM_SHARED((NPAD, dp), jnp.float32),
            pltpu.SemaphoreType.DMA((NI,)),
            pltpu.SemaphoreType.DMA((NR,)),
            pltpu.SemaphoreType.DMA((NR,)),
        ],
        compiler_params=pltpu.CompilerParams(use_tc_tiling_on_sc=False,
                                             needs_layout_passes=False),
    )
    def edge_kernel(featp, er, idxr, zv, vout,
                    idxbuf, er_t, wbuf, rows, acc, isem, gsem, ssem):
        cid = lax.axis_index("c")
        sid = lax.axis_index("s")
        wid = cid * NS + sid

        def issue_idx(ci):
            bi = lax.rem(ci, NI)
            pltpu.async_copy(idxr.at[wid, ci], idxbuf.at[bi], isem.at[bi])

        def wait_idx(ci):
            bi = lax.rem(ci, NI)
            pltpu.make_async_copy(idxr.at[wid, 0], idxbuf.at[bi],
                                  isem.at[bi]).wait()

        def issue_gather(ci):
            b = lax.rem(ci, NR)
            bi = lax.rem(ci, NI)
            pltpu.async_copy(featp.at[idxbuf.at[bi, 0]], rows.at[b],
                             gsem.at[b])

        def wait_gather(ci):
            b = lax.rem(ci, NR)
            pltpu.make_async_copy(featp.at[pl.ds(0, C)], rows.at[b],
                                  gsem.at[b]).wait()

        def issue_scatter(ci):
            b = lax.rem(ci, NR)
            bi = lax.rem(ci, NI)
            pltpu.async_copy(rows.at[b], acc.at[idxbuf.at[bi, 1]], ssem.at[b],
                             add=True)

        def wait_scatter(ci):
            b = lax.rem(ci, NR)
            pltpu.make_async_copy(featp.at[pl.ds(0, C)], rows.at[b],
                                  ssem.at[b]).wait()

        pltpu.sync_copy(er, er_t)
        pltpu.sync_copy(zv, acc.at[pl.ds(sid * NPT, NPT)])
        issue_idx(0)
        issue_idx(1)
        wait_idx(0)
        issue_gather(0)
        plsc.subcore_barrier()

        def chunk_body(ci, carry):
            b = lax.rem(ci, NR)
            bi = lax.rem(ci, NI)

            @pl.when(ci >= 1)
            def _():
                wait_scatter(ci - 1)

            @pl.when(ci + 2 < CHUNKS)
            def _():
                issue_idx(ci + 2)

            @pl.when(ci + 1 < CHUNKS)
            def _():
                wait_idx(ci + 1)
                issue_gather(ci + 1)

            wait_gather(ci)
            bv = b + jnp.zeros((16,), jnp.int32)
            cv = jnp.full((16,), odim + 1, jnp.int32)
            for g in range(C // 16):
                dv = idxbuf[bi, 1, pl.ds(g * 16, 16)]
                erv = plsc.load_gather(er_t, [dv])
                jv = lax.broadcasted_iota(jnp.int32, (16,), 0) + g * 16
                elv = plsc.load_gather(rows, [bv, jv, cv])
                ev = elv + erv
                ev = jnp.where(ev > 0, ev, SLOPE * ev)
                wbuf[pl.ds(g * 16, 16)] = jnp.exp(ev)

            for g in range(C // 16):
                w16 = wbuf[pl.ds(g * 16, 16)]
                for jj in range(16):
                    a = w16[jj]
                    j = g * 16 + jj
                    for k in range(kv):
                        rows[b, j, pl.ds(k * 16, 16)] = (
                            rows[b, j, pl.ds(k * 16, 16)] * a)

            issue_scatter(ci)
            return carry

        lax.fori_loop(0, CHUNKS, chunk_body, 0)
        wait_scatter(CHUNKS - 1)
        plsc.subcore_barrier()
        pltpu.sync_copy(acc.at[pl.ds(sid * NPT, NPT)],
                        vout.at[cid, pl.ds(sid * NPT, NPT)])

    return edge_kernel


def _edge_aggregate(featp, er, src, dst, dp, odim):
    zv = jnp.zeros((NPT, dp), jnp.float32)
    idxr = jnp.stack([src.reshape(NW, CHUNKS, C),
                      dst.reshape(NW, CHUNKS, C)], axis=2)
    return _make_edge_kernel(dp, odim)(featp, er, idxr, zv)


def _pad_params(W, al, ar, dp):
    odim = W.shape[1]
    Wp = jnp.pad(W, ((0, 0), (0, dp - odim)))
    alp = jnp.pad(al, ((0, 0), (0, dp - odim)))
    arp = jnp.pad(ar, ((0, 0), (0, dp - odim)))
    return Wp, alp, arp


def kernel(x, edge_index, W0, al0, ar0, W1, al1, ar1, W2, al2, ar2):
    src = edge_index[0]
    dst = edge_index[1]

    Wp0, alp0, arp0 = _pad_params(W0, al0, ar0, 144)
    Wp1, alp1, arp1 = _pad_params(W1, al1, ar1, 144)
    Wp2, alp2, arp2 = _pad_params(W2, al2, ar2, 64)

    featp, er = _dense(x, Wp0, alp0, arp0, odim=128, dp=144)
    vout = _edge_aggregate(featp, er.reshape(N), src, dst, 144, 128)

    featp, er = _combine_dense(vout, Wp1, alp1, arp1, prev=128, odim=128,
                               dp=144)
    vout = _edge_aggregate(featp, er.reshape(N), src, dst, 144, 128)

    featp, er = _combine_dense(vout, Wp2, alp2, arp2, prev=128, odim=40,
                               dp=64)
    vout = _edge_aggregate(featp, er.reshape(N), src, dst, 64, 40)

    return _final(vout, odim=40)


# speedup vs baseline: 1.0797x; 1.0797x over previous
import functools

import jax
import jax.numpy as jnp
from jax import lax
from jax.experimental import pallas as pl
from jax.experimental.pallas import tpu as pltpu
from jax.experimental.pallas import tpu_sc as plsc

N = 10000
E = 320000
SLOPE = 0.2
EPS = 1e-9

NC = 2
NS = 16
NW = NC * NS
EPT = E // NW
C = 80
CHUNKS = EPT // C
NPAD = 10240
NPT = NPAD // NS

NR = 3
NI = 4


def _embed(featp, el, odim):
    col = lax.broadcasted_iota(jnp.int32, featp.shape, 1)
    return jnp.where(col == odim, 1.0, jnp.where(col == odim + 1, el, featp))


def _dense_body(h_ref, wp_ref, alp_ref, arp_ref, featp_ref, er_ref, *, odim):
    featp = jnp.dot(h_ref[...], wp_ref[...], preferred_element_type=jnp.float32)
    el = jnp.sum(featp * alp_ref[...], axis=1, keepdims=True)
    er_ref[...] = jnp.sum(featp * arp_ref[...], axis=1, keepdims=True)
    featp_ref[...] = _embed(featp, el, odim)


def _combine_dense_body(vout_ref, wp_ref, alp_ref, arp_ref,
                        featp_ref, er_ref, *, prev, odim):
    tot = vout_ref[0][:N] + vout_ref[1][:N]
    v = tot[:, :prev]
    s = tot[:, prev:prev + 1]
    h = jnp.maximum(v / (s + EPS), 0.0)
    featp = jnp.dot(h, wp_ref[...], preferred_element_type=jnp.float32)
    el = jnp.sum(featp * alp_ref[...], axis=1, keepdims=True)
    er_ref[...] = jnp.sum(featp * arp_ref[...], axis=1, keepdims=True)
    featp_ref[...] = _embed(featp, el, odim)


def _final_body(vout_ref, out_ref, *, odim):
    tot = vout_ref[0][:N] + vout_ref[1][:N]
    out_ref[...] = tot[:, :odim] / (tot[:, odim:odim + 1] + EPS)


def _dense(h, Wp, alp, arp, odim, dp):
    return pl.pallas_call(
        functools.partial(_dense_body, odim=odim),
        out_shape=[
            jax.ShapeDtypeStruct((N, dp), jnp.float32),
            jax.ShapeDtypeStruct((N, 1), jnp.float32),
        ],
    )(h, Wp, alp, arp)


def _combine_dense(vout, Wp, alp, arp, prev, odim, dp):
    return pl.pallas_call(
        functools.partial(_combine_dense_body, prev=prev, odim=odim),
        out_shape=[
            jax.ShapeDtypeStruct((N, dp), jnp.float32),
            jax.ShapeDtypeStruct((N, 1), jnp.float32),
        ],
    )(vout, Wp, alp, arp)


def _final(vout, odim):
    return pl.pallas_call(
        functools.partial(_final_body, odim=odim),
        out_shape=jax.ShapeDtypeStruct((N, odim), jnp.float32),
    )(vout)


@functools.lru_cache(maxsize=None)
def _make_edge_kernel(dp, odim):
    kv = dp // 16
    mesh = plsc.VectorSubcoreMesh(core_axis_name="c", subcore_axis_name="s")

    @functools.partial(
        pl.kernel,
        out_type=jax.ShapeDtypeStruct((NC, NPAD, dp), jnp.float32),
        mesh=mesh,
        scratch_types=[
            pltpu.VMEM((NI, 2, C), jnp.int32),
            pltpu.VMEM((NR, C), jnp.float32),
            pltpu.VMEM((C,), jnp.float32),
            pltpu.VMEM((NR, C, dp), jnp.float32),
            pltpu.VMEM_SHARED((NPAD, dp), jnp.float32),
            pltpu.SemaphoreType.DMA((NI,)),
            pltpu.SemaphoreType.DMA((NR,)),
            pltpu.SemaphoreType.DMA((NR,)),
        ],
        compiler_params=pltpu.CompilerParams(use_tc_tiling_on_sc=False,
                                             needs_layout_passes=False),
    )
    def edge_kernel(featp, er, idxr, zv, vout,
                    idxbuf, erbuf, wbuf, rows, acc,
                    isem, gsem, ssem):
        cid = lax.axis_index("c")
        sid = lax.axis_index("s")
        wid = cid * NS + sid

        def issue_idx(ci):
            bi = lax.rem(ci, NI)
            pltpu.async_copy(idxr.at[wid, ci], idxbuf.at[bi], isem.at[bi])

        def wait_idx(ci):
            bi = lax.rem(ci, NI)
            pltpu.make_async_copy(idxr.at[wid, 0], idxbuf.at[bi],
                                  isem.at[bi]).wait()

        def issue_gather(ci):
            b = lax.rem(ci, NR)
            bi = lax.rem(ci, NI)
            pltpu.async_copy(featp.at[idxbuf.at[bi, 0]], rows.at[b],
                             gsem.at[b])
            pltpu.async_copy(er.at[idxbuf.at[bi, 1]], erbuf.at[b],
                             gsem.at[b])

        def wait_gather(ci):
            b = lax.rem(ci, NR)
            pltpu.make_async_copy(featp.at[pl.ds(0, C)], rows.at[b],
                                  gsem.at[b]).wait()
            pltpu.make_async_copy(er.at[pl.ds(0, C)], erbuf.at[b],
                                  gsem.at[b]).wait()

        def issue_scatter(ci):
            b = lax.rem(ci, NR)
            bi = lax.rem(ci, NI)
            pltpu.async_copy(rows.at[b], acc.at[idxbuf.at[bi, 1]], ssem.at[b],
                             add=True)

        def wait_scatter(ci):
            b = lax.rem(ci, NR)
            pltpu.make_async_copy(featp.at[pl.ds(0, C)], rows.at[b],
                                  ssem.at[b]).wait()

        pltpu.sync_copy(zv, acc.at[pl.ds(sid * NPT, NPT)])
        issue_idx(0)
        issue_idx(1)
        wait_idx(0)
        issue_gather(0)
        plsc.subcore_barrier()

        def chunk_body(ci, carry):
            b = lax.rem(ci, NR)
            bi = lax.rem(ci, NI)

            @pl.when(ci >= 2)
            def _():
                wait_scatter(ci - 2)

            @pl.when(ci + 2 < CHUNKS)
            def _():
                issue_idx(ci + 2)

            @pl.when(ci + 1 < CHUNKS)
            def _():
                wait_idx(ci + 1)
                issue_gather(ci + 1)

            wait_gather(ci)
            bv = b + jnp.zeros((16,), jnp.int32)
            cv = jnp.full((16,), odim + 1, jnp.int32)
            for g in range(C // 16):
                erv = erbuf[b, pl.ds(g * 16, 16)]
                jv = lax.broadcasted_iota(jnp.int32, (16,), 0) + g * 16
                elv = plsc.load_gather(rows, [bv, jv, cv])
                ev = elv + erv
                ev = jnp.where(ev > 0, ev, SLOPE * ev)
                wbuf[pl.ds(g * 16, 16)] = jnp.exp(ev)

            for g in range(C // 16):
                w16 = wbuf[pl.ds(g * 16, 16)]
                for jj in range(16):
                    a = w16[jj]
                    j = g * 16 + jj
                    for k in range(kv):
                        rows[b, j, pl.ds(k * 16, 16)] = (
                            rows[b, j, pl.ds(k * 16, 16)] * a)

            issue_scatter(ci)
            return carry

        lax.fori_loop(0, CHUNKS, chunk_body, 0)
        wait_scatter(CHUNKS - 2)
        wait_scatter(CHUNKS - 1)
        plsc.subcore_barrier()
        pltpu.sync_copy(acc.at[pl.ds(sid * NPT, NPT)],
                        vout.at[cid, pl.ds(sid * NPT, NPT)])

    return edge_kernel


def _edge_aggregate(featp, er, src, dst, dp, odim):
    zv = jnp.zeros((NPT, dp), jnp.float32)
    idxr = jnp.stack([src.reshape(NW, CHUNKS, C),
                      dst.reshape(NW, CHUNKS, C)], axis=2)
    return _make_edge_kernel(dp, odim)(featp, er, idxr, zv)


def _pad_params(W, al, ar, dp):
    odim = W.shape[1]
    Wp = jnp.pad(W, ((0, 0), (0, dp - odim)))
    alp = jnp.pad(al, ((0, 0), (0, dp - odim)))
    arp = jnp.pad(ar, ((0, 0), (0, dp - odim)))
    return Wp, alp, arp


def kernel(x, edge_index, W0, al0, ar0, W1, al1, ar1, W2, al2, ar2):
    src = edge_index[0]
    dst = edge_index[1]

    Wp0, alp0, arp0 = _pad_params(W0, al0, ar0, 144)
    Wp1, alp1, arp1 = _pad_params(W1, al1, ar1, 144)
    Wp2, alp2, arp2 = _pad_params(W2, al2, ar2, 64)

    featp, er = _dense(x, Wp0, alp0, arp0, odim=128, dp=144)
    vout = _edge_aggregate(featp, er.reshape(N), src, dst, 144, 128)

    featp, er = _combine_dense(vout, Wp1, alp1, arp1, prev=128, odim=128,
                               dp=144)
    vout = _edge_aggregate(featp, er.reshape(N), src, dst, 144, 128)

    featp, er = _combine_dense(vout, Wp2, alp2, arp2, prev=128, odim=40,
                               dp=64)
    vout = _edge_aggregate(featp, er.reshape(N), src, dst, 64, 40)

    return _final(vout, odim=40)


# speedup vs baseline: 1.0985x; 1.0174x over previous
import functools

import jax
import jax.numpy as jnp
from jax import lax
from jax.experimental import pallas as pl
from jax.experimental.pallas import tpu as pltpu
from jax.experimental.pallas import tpu_sc as plsc

N = 10000
E = 320000
SLOPE = 0.2
EPS = 1e-9

NC = 2
NS = 16
NW = NC * NS
EPT = E // NW
C = 80
CHUNKS = EPT // C
NPAD = 10240
NPT = NPAD // NS

NR = 3
NI = 4


def _embed(featp, el, odim):
    col = lax.broadcasted_iota(jnp.int32, featp.shape, 1)
    return jnp.where(col == odim, 1.0, jnp.where(col == odim + 1, el, featp))


def _dense_body(h_ref, wp_ref, alp_ref, arp_ref, featp_ref, er_ref, *, odim):
    featp = jnp.dot(h_ref[...], wp_ref[...], preferred_element_type=jnp.float32)
    el = jnp.sum(featp * alp_ref[...], axis=1, keepdims=True)
    er_ref[...] = jnp.sum(featp * arp_ref[...], axis=1, keepdims=True)
    featp_ref[...] = _embed(featp, el, odim)


def _combine_dense_body(vout_ref, wp_ref, alp_ref, arp_ref,
                        featp_ref, er_ref, *, prev, odim):
    tot = vout_ref[0][:N] + vout_ref[1][:N]
    v = tot[:, :prev]
    s = tot[:, prev:prev + 1]
    h = jnp.maximum(v / (s + EPS), 0.0)
    featp = jnp.dot(h, wp_ref[...], preferred_element_type=jnp.float32)
    el = jnp.sum(featp * alp_ref[...], axis=1, keepdims=True)
    er_ref[...] = jnp.sum(featp * arp_ref[...], axis=1, keepdims=True)
    featp_ref[...] = _embed(featp, el, odim)


def _final_body(vout_ref, out_ref, *, odim):
    tot = vout_ref[0][:N] + vout_ref[1][:N]
    out_ref[...] = tot[:, :odim] / (tot[:, odim:odim + 1] + EPS)


def _dense(h, Wp, alp, arp, odim, dp):
    return pl.pallas_call(
        functools.partial(_dense_body, odim=odim),
        out_shape=[
            jax.ShapeDtypeStruct((N, dp), jnp.float32),
            jax.ShapeDtypeStruct((N, 1), jnp.float32),
        ],
    )(h, Wp, alp, arp)


def _combine_dense(vout, Wp, alp, arp, prev, odim, dp):
    return pl.pallas_call(
        functools.partial(_combine_dense_body, prev=prev, odim=odim),
        out_shape=[
            jax.ShapeDtypeStruct((N, dp), jnp.float32),
            jax.ShapeDtypeStruct((N, 1), jnp.float32),
        ],
    )(vout, Wp, alp, arp)


def _final(vout, odim):
    return pl.pallas_call(
        functools.partial(_final_body, odim=odim),
        out_shape=jax.ShapeDtypeStruct((N, odim), jnp.float32),
    )(vout)


@functools.lru_cache(maxsize=None)
def _make_edge_kernel(dp, odim):
    kv = dp // 16
    mesh = plsc.VectorSubcoreMesh(core_axis_name="c", subcore_axis_name="s")

    @functools.partial(
        pl.kernel,
        out_type=jax.ShapeDtypeStruct((NC, NPAD, dp), jnp.float32),
        mesh=mesh,
        scratch_types=[
            pltpu.VMEM((NI, 2, C), jnp.int32),
            pltpu.VMEM((NR, C), jnp.float32),
            pltpu.VMEM((NR, C, dp), jnp.float32),
            pltpu.VMEM_SHARED((NPAD, dp), jnp.float32),
            pltpu.SemaphoreType.DMA((NI,)),
            pltpu.SemaphoreType.DMA((NR,)),
            pltpu.SemaphoreType.DMA((NR,)),
        ],
        compiler_params=pltpu.CompilerParams(use_tc_tiling_on_sc=False,
                                             needs_layout_passes=False),
    )
    def edge_kernel(featp, er, idxr, zv, vout,
                    idxbuf, erbuf, rows, acc,
                    isem, gsem, ssem):
        cid = lax.axis_index("c")
        sid = lax.axis_index("s")
        wid = cid * NS + sid

        def issue_idx(ci):
            bi = lax.rem(ci, NI)
            pltpu.async_copy(idxr.at[wid, ci], idxbuf.at[bi], isem.at[bi])

        def wait_idx(ci):
            bi = lax.rem(ci, NI)
            pltpu.make_async_copy(idxr.at[wid, 0], idxbuf.at[bi],
                                  isem.at[bi]).wait()

        def issue_gather(ci):
            b = lax.rem(ci, NR)
            bi = lax.rem(ci, NI)
            pltpu.async_copy(featp.at[idxbuf.at[bi, 0]], rows.at[b],
                             gsem.at[b])
            pltpu.async_copy(er.at[idxbuf.at[bi, 1]], erbuf.at[b],
                             gsem.at[b])

        def wait_gather(ci):
            b = lax.rem(ci, NR)
            pltpu.make_async_copy(featp.at[pl.ds(0, C)], rows.at[b],
                                  gsem.at[b]).wait()
            pltpu.make_async_copy(er.at[pl.ds(0, C)], erbuf.at[b],
                                  gsem.at[b]).wait()

        def issue_scatter(ci):
            b = lax.rem(ci, NR)
            bi = lax.rem(ci, NI)
            pltpu.async_copy(rows.at[b], acc.at[idxbuf.at[bi, 1]], ssem.at[b],
                             add=True)

        def wait_scatter(ci):
            b = lax.rem(ci, NR)
            pltpu.make_async_copy(featp.at[pl.ds(0, C)], rows.at[b],
                                  ssem.at[b]).wait()

        pltpu.sync_copy(zv, acc.at[pl.ds(sid * NPT, NPT)])
        issue_idx(0)
        issue_idx(1)
        wait_idx(0)
        issue_gather(0)
        plsc.subcore_barrier()

        def chunk_body(ci, carry):
            b = lax.rem(ci, NR)
            bi = lax.rem(ci, NI)

            @pl.when(ci >= 2)
            def _():
                wait_scatter(ci - 2)

            @pl.when(ci + 2 < CHUNKS)
            def _():
                issue_idx(ci + 2)

            @pl.when(ci + 1 < CHUNKS)
            def _():
                wait_idx(ci + 1)
                issue_gather(ci + 1)

            wait_gather(ci)
            bv = b + jnp.zeros((16,), jnp.int32)
            cv = jnp.full((16,), odim + 1, jnp.int32)
            for g in range(C // 16):
                erv = erbuf[b, pl.ds(g * 16, 16)]
                jv = lax.broadcasted_iota(jnp.int32, (16,), 0) + g * 16
                elv = plsc.load_gather(rows, [bv, jv, cv])
                ev = elv + erv
                ev = jnp.where(ev > 0, ev, SLOPE * ev)
                w16 = jnp.exp(ev)
                for jj in range(16):
                    a = w16[jj]
                    j = g * 16 + jj
                    for k in range(kv):
                        rows[b, j, pl.ds(k * 16, 16)] = (
                            rows[b, j, pl.ds(k * 16, 16)] * a)

            issue_scatter(ci)
            return carry

        lax.fori_loop(0, CHUNKS, chunk_body, 0)
        wait_scatter(CHUNKS - 2)
        wait_scatter(CHUNKS - 1)
        plsc.subcore_barrier()
        pltpu.sync_copy(acc.at[pl.ds(sid * NPT, NPT)],
                        vout.at[cid, pl.ds(sid * NPT, NPT)])

    return edge_kernel


def _edge_aggregate(featp, er, src, dst, dp, odim):
    zv = jnp.zeros((NPT, dp), jnp.float32)
    idxr = jnp.stack([src.reshape(NW, CHUNKS, C),
                      dst.reshape(NW, CHUNKS, C)], axis=2)
    return _make_edge_kernel(dp, odim)(featp, er, idxr, zv)


def _pad_params(W, al, ar, dp):
    odim = W.shape[1]
    Wp = jnp.pad(W, ((0, 0), (0, dp - odim)))
    alp = jnp.pad(al, ((0, 0), (0, dp - odim)))
    arp = jnp.pad(ar, ((0, 0), (0, dp - odim)))
    return Wp, alp, arp


def kernel(x, edge_index, W0, al0, ar0, W1, al1, ar1, W2, al2, ar2):
    src = edge_index[0]
    dst = edge_index[1]

    Wp0, alp0, arp0 = _pad_params(W0, al0, ar0, 144)
    Wp1, alp1, arp1 = _pad_params(W1, al1, ar1, 144)
    Wp2, alp2, arp2 = _pad_params(W2, al2, ar2, 48)

    featp, er = _dense(x, Wp0, alp0, arp0, odim=128, dp=144)
    vout = _edge_aggregate(featp, er.reshape(N), src, dst, 144, 128)

    featp, er = _combine_dense(vout, Wp1, alp1, arp1, prev=128, odim=128,
                               dp=144)
    vout = _edge_aggregate(featp, er.reshape(N), src, dst, 144, 128)

    featp, er = _combine_dense(vout, Wp2, alp2, arp2, prev=128, odim=40,
                               dp=48)
    vout = _edge_aggregate(featp, er.reshape(N), src, dst, 48, 40)

    return _final(vout, odim=40)
